# baseline (speedup 1.0000x reference)
import dataclasses
import functools

import jax
import jax.numpy as jnp
from jax import lax
from jax.experimental import pallas as pl
from jax.experimental.pallas import tpu as pltpu, tpu_sc as plsc

N = 10000
E = 320000
NP = 10240
NC, NS = 2, 16
NW = NC * NS
C = 112
NBLK = 92
EPW = C * NBLK
EP = NW * EPW
RPT = NP // NS

_mesh = plsc.VectorSubcoreMesh(core_axis_name="c", subcore_axis_name="s")
_sc_params = dataclasses.replace(
    pltpu.CompilerParams(use_tc_tiling_on_sc=False),
    needs_layout_passes=False)


def _bcast(v16, lane):
    idx = jnp.full((16, 1), lane, dtype=jnp.int32)
    dn = lax.GatherDimensionNumbers(offset_dims=(), collapsed_slice_dims=(0,),
                                    start_index_map=(0,))
    return lax.gather(v16, idx, dn, (1,),
                      mode=lax.GatherScatterMode.PROMISE_IN_BOUNDS)


def _interleave(x):
    n, w = x.shape
    return x.reshape(n, w // 32, 2, 16).transpose(0, 1, 3, 2).reshape(n, w)


def _sc_edge_layer(he, er, edges, heads, F):
    FA = F + 16
    HCH = F // 32
    HE = 32 * HCH + 32
    D = F // heads

    @functools.partial(
        pl.kernel,
        out_type=jax.ShapeDtypeStruct((NC, NP, FA), jnp.float32),
        mesh=_mesh,
        compiler_params=_sc_params,
        scratch_types=[
            pltpu.VMEM((4, 2, C), jnp.int32),
            pltpu.VMEM((2, C, 16), jnp.float32),
            pltpu.VMEM((2, C, HE), jnp.bfloat16),
            pltpu.VMEM((C, FA), jnp.float32),
            pltpu.VMEM_SHARED((NP, FA), jnp.float32),
            pltpu.SemaphoreType.DMA,
            pltpu.SemaphoreType.DMA,
            pltpu.SemaphoreType.DMA,
            pltpu.SemaphoreType.DMA,
            pltpu.SemaphoreType.DMA,
            pltpu.SemaphoreType.DMA,
            pltpu.SemaphoreType.DMA,
        ],
    )
    def k(he_hbm, er_hbm, edges_hbm, acc_out,
          idx, B, Hbf, msg, acc_sh, i0, i1, i2, i3, g0, g1, ssem):
        cid = lax.axis_index("c")
        sid = lax.axis_index("s")
        wid = cid * NS + sid
        r0 = sid * RPT
        isem = (i0, i1, i2, i3)
        gsem = (g0, g1)
        base = wid * EPW

        @pl.loop(0, C)
        def _(i):
            for j in range(FA // 16):
                msg[i, pl.ds(j * 16, 16)] = jnp.zeros((16,), jnp.float32)

        nfull, rem = divmod(RPT, C)
        for j in range(nfull):
            pltpu.sync_copy(msg, acc_sh.at[pl.ds(r0 + j * C, C)])
        if rem:
            pltpu.sync_copy(msg.at[pl.ds(0, rem)],
                            acc_sh.at[pl.ds(r0 + nfull * C, rem)])
        plsc.subcore_barrier()

        def issue_idx(g, ib):
            pltpu.async_copy(edges_hbm.at[:, pl.ds(base + g * C, C)], idx.at[ib], isem[ib])

        def wait_idx(ib):
            pltpu.make_async_copy(edges_hbm.at[:, pl.ds(0, C)], idx.at[ib], isem[ib]).wait()

        def issue_gathers(b, ib):
            pltpu.async_copy(er_hbm.at[idx.at[ib, 1]], B.at[b], gsem[b])
            pltpu.async_copy(he_hbm.at[idx.at[ib, 0]], Hbf.at[b], gsem[b])

        def wait_gathers(b, ib):
            pltpu.make_async_copy(er_hbm.at[idx.at[ib, 1]], B.at[b], gsem[b]).wait()
            pltpu.make_async_copy(he_hbm.at[idx.at[ib, 0]], Hbf.at[b], gsem[b]).wait()

        def issue_scatter(ib):
            pltpu.async_copy(msg, acc_sh.at[idx.at[ib, 1]], ssem, add=True)

        def wait_scatter(ib):
            pltpu.make_async_copy(msg, acc_sh.at[idx.at[ib, 1]], ssem).wait()

        def compute(b):
            @pl.loop(0, C)
            def _(i):
                ev = Hbf[b, i, pl.ds(HCH * 32, 32)]
                el_f, _ = plsc.unpack(ev, format=plsc.PackFormat.INTERLEAVED)
                z = el_f + B[b, i]
                sv = jnp.exp(jnp.maximum(z, 0.2 * z))
                msg[i, pl.ds(F, 16)] = sv
                for q in range(HCH):
                    v = Hbf[b, i, pl.ds(32 * q, 32)]
                    c0, c1 = plsc.unpack(v, format=plsc.PackFormat.INTERLEAVED)
                    bc0 = _bcast(sv, (32 * q) // D)
                    bc1 = _bcast(sv, (32 * q + 16) // D)
                    msg[i, pl.ds(32 * q, 16)] = c0 * bc0
                    msg[i, pl.ds(32 * q + 16, 16)] = c1 * bc1

        issue_idx(0, 0)
        issue_idx(1, 1)
        wait_idx(0)
        issue_gathers(0, 0)

        def body(g, k_):
            b = k_ % 2
            bp = (k_ + 1) % 2
            ib1 = (k_ + 1) % 4
            ib2 = (k_ + 2) % 4
            ibp = (k_ + 3) % 4

            @pl.when(g + 1 < NBLK)
            def _():
                wait_idx(ib1)
                issue_gathers(bp, ib1)

            @pl.when(g + 2 < NBLK)
            def _():
                issue_idx(g + 2, ib2)

            wait_gathers(b, k_)

            @pl.when(g >= 1)
            def _():
                wait_scatter(ibp)

            compute(b)
            issue_scatter(k_)

        @pl.loop(0, NBLK // 4)
        def _(p):
            for k_ in range(4):
                body(4 * p + k_, k_)

        wait_scatter((NBLK - 1) % 4)
        plsc.subcore_barrier()
        pltpu.sync_copy(acc_sh.at[pl.ds(r0, RPT)], acc_out.at[cid, pl.ds(r0, RPT)])

    return k(he, er, edges)


def _dense1_body(x_ref, W_ref, al_ref, ar_ref, he_ref, er_ref, *, heads, out_dim):
    x = x_ref[...]
    n = x.shape[0]
    h = lax.dot(x, W_ref[...], precision=lax.Precision.HIGHEST)
    hr = h.reshape(n, heads, out_dim)
    el = jnp.sum(hr * al_ref[...][None], axis=-1)
    er = jnp.sum(hr * ar_ref[...][None], axis=-1)
    pad = jnp.zeros((n, 16 - heads), jnp.float32)
    el16 = jnp.concatenate([el, pad], axis=1)
    elz = jnp.stack([el16, jnp.zeros_like(el16)], axis=-1).reshape(n, 32)
    he = jnp.concatenate([_interleave(h), elz], axis=1)
    he_ref[...] = he.astype(jnp.bfloat16)
    er_ref[...] = jnp.concatenate([er, pad], axis=1)


NB = 1000


def _dense1(x, W, al, ar, heads, out_dim):
    n = x.shape[0]
    d_in = x.shape[1]
    body = functools.partial(_dense1_body, heads=heads, out_dim=out_dim)
    return pl.pallas_call(
        body,
        grid=(n // NB,),
        in_specs=[
            pl.BlockSpec((NB, d_in), lambda i: (i, 0)),
            pl.BlockSpec((d_in, heads * out_dim), lambda i: (0, 0)),
            pl.BlockSpec((heads, out_dim), lambda i: (0, 0)),
            pl.BlockSpec((heads, out_dim), lambda i: (0, 0)),
        ],
        out_specs=(
            pl.BlockSpec((NB, heads * out_dim + 32), lambda i: (i, 0)),
            pl.BlockSpec((NB, 16), lambda i: (i, 0)),
        ),
        out_shape=(
            jax.ShapeDtypeStruct((n, heads * out_dim + 32), jnp.bfloat16),
            jax.ShapeDtypeStruct((n, 16), jnp.float32),
        ),
    )(x, W, al, ar)


def _epi1_body(acc_ref, b_ref, W2_ref, al2_ref, ar2_ref, he2_ref, er2_ref):
    acc = acc_ref[0, :, :128] + acc_ref[1, :, :128]
    den = acc_ref[0, :, 128:136] + acc_ref[1, :, 128:136]
    val = acc.reshape(NB, 8, 16) / (den[:, :, None] + 1e-9)
    out1 = jnp.maximum(val.reshape(NB, 128) + b_ref[...][None, :], 0.0)
    h2 = lax.dot(out1, W2_ref[...], precision=lax.Precision.HIGHEST)
    hr = h2.reshape(NB, 1, 32)
    el2 = jnp.sum(hr * al2_ref[...][None], axis=-1)
    er2 = jnp.sum(hr * ar2_ref[...][None], axis=-1)
    pad = jnp.zeros((NB, 15), jnp.float32)
    el16 = jnp.concatenate([el2, pad], axis=1)
    elz = jnp.stack([el16, jnp.zeros_like(el16)], axis=-1).reshape(NB, 32)
    he2 = jnp.concatenate([_interleave(h2), elz], axis=1)
    he2_ref[...] = he2.astype(jnp.bfloat16)
    er2_ref[...] = jnp.concatenate([er2, pad], axis=1)


def _epi2_body(acc_ref, b_ref, out_ref):
    acc = acc_ref[0, :, :32] + acc_ref[1, :, :32]
    den = acc_ref[0, :, 32:33] + acc_ref[1, :, 32:33]
    val = acc / (den + 1e-9) + b_ref[...][None, :]
    m = jnp.max(val, axis=1, keepdims=True)
    ex = jnp.exp(val - m)
    out_ref[...] = val - m - jnp.log(jnp.sum(ex, axis=1, keepdims=True))


def kernel(inputs, edge_index, W1, al1, ar1, b1, W2, al2, ar2, b2):
    pad_src = jnp.zeros((EP - E,), jnp.int32)
    pad_dst = N + (jnp.arange(EP - E, dtype=jnp.int32) % (NP - N))
    edges = jnp.concatenate([edge_index, jnp.stack([pad_src, pad_dst])], axis=1)

    he1, er1 = _dense1(inputs, W1, al1, ar1, 8, 16)
    er1 = jnp.pad(er1, ((0, NP - N), (0, 0)))
    acc1 = _sc_edge_layer(he1, er1, edges, 8, 128)

    he2, er2 = pl.pallas_call(
        _epi1_body,
        grid=(N // NB,),
        in_specs=[
            pl.BlockSpec((2, NB, 144), lambda i: (0, i, 0)),
            pl.BlockSpec((128,), lambda i: (0,)),
            pl.BlockSpec((128, 32), lambda i: (0, 0)),
            pl.BlockSpec((1, 32), lambda i: (0, 0)),
            pl.BlockSpec((1, 32), lambda i: (0, 0)),
        ],
        out_specs=(
            pl.BlockSpec((NB, 64), lambda i: (i, 0)),
            pl.BlockSpec((NB, 16), lambda i: (i, 0)),
        ),
        out_shape=(
            jax.ShapeDtypeStruct((N, 64), jnp.bfloat16),
            jax.ShapeDtypeStruct((N, 16), jnp.float32),
        ),
    )(acc1, b1, W2, al2, ar2)

    er2 = jnp.pad(er2, ((0, NP - N), (0, 0)))
    acc2 = _sc_edge_layer(he2, er2, edges, 1, 32)

    out = pl.pallas_call(
        _epi2_body,
        grid=(N // NB,),
        in_specs=[
            pl.BlockSpec((2, NB, 48), lambda i: (0, i, 0)),
            pl.BlockSpec((32,), lambda i: (0,)),
        ],
        out_specs=pl.BlockSpec((NB, 32), lambda i: (i, 0)),
        out_shape=jax.ShapeDtypeStruct((N, 32), jnp.float32),
    )(acc2, b2)
    return out


# speedup vs baseline: 1.2424x; 1.2424x over previous
import dataclasses
import functools

import jax
import jax.numpy as jnp
from jax import lax
from jax.experimental import pallas as pl
from jax.experimental.pallas import tpu as pltpu, tpu_sc as plsc

N = 10000
E = 320000
NP = 10240
NC, NS = 2, 16
NW = NC * NS
C = 112
NBLK = 92
EPW = C * NBLK
EP = NW * EPW
RPT = NP // NS

_mesh = plsc.VectorSubcoreMesh(core_axis_name="c", subcore_axis_name="s")
_sc_params = dataclasses.replace(
    pltpu.CompilerParams(use_tc_tiling_on_sc=False),
    needs_layout_passes=False)


def _gat(v16, idx16):
    dn = lax.GatherDimensionNumbers(offset_dims=(), collapsed_slice_dims=(0,),
                                    start_index_map=(0,))
    return lax.gather(v16, idx16[:, None], dn, (1,),
                      mode=lax.GatherScatterMode.PROMISE_IN_BOUNDS)


def _bcast(v16, lane):
    return _gat(v16, jnp.full((16,), lane, dtype=jnp.int32))


import numpy as _np

def _mk_perm(width):
    p = _np.empty(width, _np.int32)
    for c in range(width):
        q, r = divmod(c, 32)
        p[c] = 32 * q + (2 * r if r < 16 else 2 * (r - 16) + 1)
    return p

_PERM1 = _mk_perm(128)
_PERM2 = _mk_perm(32)
_MSEL = _np.zeros((16, 128), _np.float32)
for _c in range(128):
    _hd = _PERM1[_c] // 16
    _MSEL[(_hd // 2) if _hd % 2 == 0 else (8 + _hd // 2), _c] = 1.0
_P32 = _np.zeros((32, 32), _np.float32)
for _c in range(32):
    _P32[_c, _PERM2[_c]] = 1.0


def _sc_edge_layer(he, er, edges, heads, F):
    FA = F + 16
    HCH = F // 32
    HE = 32 * HCH + 32
    D = F // heads

    @functools.partial(
        pl.kernel,
        out_type=jax.ShapeDtypeStruct((NC, NP, FA), jnp.float32),
        mesh=_mesh,
        compiler_params=_sc_params,
        scratch_types=[
            pltpu.VMEM((4, 2, C), jnp.int32),
            pltpu.VMEM((2, C, 32), jnp.bfloat16),
            pltpu.VMEM((2, C, HE), jnp.bfloat16),
            pltpu.VMEM((C, FA), jnp.float32),
            pltpu.VMEM_SHARED((NP, FA), jnp.float32),
            pltpu.SemaphoreType.DMA,
            pltpu.SemaphoreType.DMA,
            pltpu.SemaphoreType.DMA,
            pltpu.SemaphoreType.DMA,
            pltpu.SemaphoreType.DMA,
            pltpu.SemaphoreType.DMA,
            pltpu.SemaphoreType.DMA,
        ],
    )
    def k(he_hbm, er_hbm, edges_hbm, acc_out,
          idx, B, Hbf, msg, acc_sh, i0, i1, i2, i3, g0, g1, ssem):
        cid = lax.axis_index("c")
        sid = lax.axis_index("s")
        wid = cid * NS + sid
        r0 = sid * RPT
        isem = (i0, i1, i2, i3)
        gsem = (g0, g1)
        base = wid * EPW

        @pl.loop(0, C)
        def _(i):
            for j in range(FA // 16):
                msg[i, pl.ds(j * 16, 16)] = jnp.zeros((16,), jnp.float32)

        nfull, rem = divmod(RPT, C)
        for j in range(nfull):
            pltpu.sync_copy(msg, acc_sh.at[pl.ds(r0 + j * C, C)])
        if rem:
            pltpu.sync_copy(msg.at[pl.ds(0, rem)],
                            acc_sh.at[pl.ds(r0 + nfull * C, rem)])
        plsc.subcore_barrier()

        def issue_idx(g, ib):
            pltpu.async_copy(edges_hbm.at[:, pl.ds(base + g * C, C)], idx.at[ib], isem[ib])

        def wait_idx(ib):
            pltpu.make_async_copy(edges_hbm.at[:, pl.ds(0, C)], idx.at[ib], isem[ib]).wait()

        def issue_gathers(b, ib):
            pltpu.async_copy(er_hbm.at[idx.at[ib, 1]], B.at[b], gsem[b])
            pltpu.async_copy(he_hbm.at[idx.at[ib, 0]], Hbf.at[b], gsem[b])

        def wait_gathers(b, ib):
            pltpu.make_async_copy(er_hbm.at[idx.at[ib, 1]], B.at[b], gsem[b]).wait()
            pltpu.make_async_copy(he_hbm.at[idx.at[ib, 0]], Hbf.at[b], gsem[b]).wait()

        def issue_scatter(ib):
            pltpu.async_copy(msg, acc_sh.at[idx.at[ib, 1]], ssem, add=True)

        def wait_scatter(ib):
            pltpu.make_async_copy(msg, acc_sh.at[idx.at[ib, 1]], ssem).wait()

        def compute(b):
            lanes = lax.iota(jnp.int32, 16)
            lo = lanes < 8
            lanes_m8 = jnp.maximum(lanes - 8, 0)

            @pl.loop(0, C)
            def _(i):
                ev = Hbf[b, i, pl.ds(HCH * 32, 32)]
                el0, el1 = plsc.unpack(ev, format=plsc.PackFormat.INTERLEAVED)
                er0, er1 = plsc.unpack(B[b, i], format=plsc.PackFormat.INTERLEAVED)
                z0 = el0 + er0
                sv0 = jnp.exp(jnp.maximum(z0, 0.2 * z0))
                z1 = el1 + er1
                sv1 = jnp.exp(jnp.maximum(z1, 0.2 * z1))
                if heads == 1:
                    msg[i, pl.ds(F, 16)] = sv0
                else:
                    msg[i, pl.ds(F, 16)] = jnp.where(lo, sv0, _gat(sv1, lanes_m8))
                for q in range(HCH):
                    v = Hbf[b, i, pl.ds(32 * q, 32)]
                    c0, c1 = plsc.unpack(v, format=plsc.PackFormat.INTERLEAVED)
                    if heads == 1:
                        bc = _bcast(sv0, 0)
                    else:
                        bc = jnp.where(lo, _bcast(sv0, q), _bcast(sv1, q))
                    msg[i, pl.ds(32 * q, 16)] = c0 * bc
                    msg[i, pl.ds(32 * q + 16, 16)] = c1 * bc

        issue_idx(0, 0)
        issue_idx(1, 1)
        wait_idx(0)
        issue_gathers(0, 0)

        def body(g, k_):
            b = k_ % 2
            bp = (k_ + 1) % 2
            ib1 = (k_ + 1) % 4
            ib2 = (k_ + 2) % 4
            ibp = (k_ + 3) % 4

            @pl.when(g + 1 < NBLK)
            def _():
                wait_idx(ib1)
                issue_gathers(bp, ib1)

            @pl.when(g + 2 < NBLK)
            def _():
                issue_idx(g + 2, ib2)

            wait_gathers(b, k_)

            @pl.when(g >= 1)
            def _():
                wait_scatter(ibp)

            compute(b)
            issue_scatter(k_)

        @pl.loop(0, NBLK // 4)
        def _(p):
            for k_ in range(4):
                body(4 * p + k_, k_)

        wait_scatter((NBLK - 1) % 4)
        plsc.subcore_barrier()
        pltpu.sync_copy(acc_sh.at[pl.ds(r0, RPT)], acc_out.at[cid, pl.ds(r0, RPT)])

    return k(he, er, edges)


def _dense1_body(x_ref, W_ref, al_ref, ar_ref, he_ref, er_ref, *, heads, out_dim):
    x = x_ref[...]
    n = x.shape[0]
    h = lax.dot(x, W_ref[...], precision=lax.Precision.HIGHEST)
    hr = h.reshape(n, heads, out_dim)
    el = jnp.sum(hr * al_ref[...][None], axis=-1)
    er = jnp.sum(hr * ar_ref[...][None], axis=-1)
    pad = jnp.zeros((n, 16 - heads), jnp.float32)
    z16 = jnp.zeros((n, 16), jnp.float32)
    he = jnp.concatenate([h, el, pad, z16], axis=1)
    he_ref[...] = he.astype(jnp.bfloat16)
    er_ref[...] = jnp.concatenate([er, pad, z16], axis=1).astype(jnp.bfloat16)


NB = 1000


def _dense1(x, W, al, ar, heads, out_dim):
    n = x.shape[0]
    d_in = x.shape[1]
    body = functools.partial(_dense1_body, heads=heads, out_dim=out_dim)
    return pl.pallas_call(
        body,
        grid=(n // NB,),
        in_specs=[
            pl.BlockSpec((NB, d_in), lambda i: (i, 0)),
            pl.BlockSpec((d_in, heads * out_dim), lambda i: (0, 0)),
            pl.BlockSpec((heads, out_dim), lambda i: (0, 0)),
            pl.BlockSpec((heads, out_dim), lambda i: (0, 0)),
        ],
        out_specs=(
            pl.BlockSpec((NB, heads * out_dim + 32), lambda i: (i, 0)),
            pl.BlockSpec((NB, 32), lambda i: (i, 0)),
        ),
        out_shape=(
            jax.ShapeDtypeStruct((n, heads * out_dim + 32), jnp.bfloat16),
            jax.ShapeDtypeStruct((n, 32), jnp.bfloat16),
        ),
    )(x, W, al, ar)


def _epi1_body(acc_ref, bp_ref, W2p_ref, al2_ref, ar2_ref, msel_ref, he2_ref, er2_ref):
    acc = acc_ref[0, :, :128] + acc_ref[1, :, :128]
    den16 = acc_ref[0, :, 128:144] + acc_ref[1, :, 128:144]
    denb = lax.dot(den16, msel_ref[...],
                   precision=lax.Precision.HIGHEST)
    out1 = jnp.maximum(acc / (denb + 1e-9) + bp_ref[...][None, :], 0.0)
    h2 = lax.dot(out1, W2p_ref[...], precision=lax.Precision.HIGHEST)
    hr = h2.reshape(NB, 1, 32)
    el2 = jnp.sum(hr * al2_ref[...][None], axis=-1)
    er2 = jnp.sum(hr * ar2_ref[...][None], axis=-1)
    pad = jnp.zeros((NB, 15), jnp.float32)
    z16 = jnp.zeros((NB, 16), jnp.float32)
    he2 = jnp.concatenate([h2, el2, pad, z16], axis=1)
    he2_ref[...] = he2.astype(jnp.bfloat16)
    er2_ref[...] = jnp.concatenate([er2, pad, z16], axis=1).astype(jnp.bfloat16)


def _epi2_body(acc_ref, bp_ref, p32_ref, out_ref):
    acc = acc_ref[0, :, :32] + acc_ref[1, :, :32]
    den = acc_ref[0, :, 32:33] + acc_ref[1, :, 32:33]
    valp = acc / (den + 1e-9) + bp_ref[...][None, :]
    val = lax.dot(valp, p32_ref[...],
                  precision=lax.Precision.HIGHEST)
    m = jnp.max(val, axis=1, keepdims=True)
    ex = jnp.exp(val - m)
    out_ref[...] = val - m - jnp.log(jnp.sum(ex, axis=1, keepdims=True))


def kernel(inputs, edge_index, W1, al1, ar1, b1, W2, al2, ar2, b2):
    pad_src = jnp.zeros((EP - E,), jnp.int32)
    pad_dst = N + (jnp.arange(EP - E, dtype=jnp.int32) % (NP - N))
    edges = jnp.concatenate([edge_index, jnp.stack([pad_src, pad_dst])], axis=1)

    b1p = b1[jnp.asarray(_PERM1)]
    W2p = W2[jnp.asarray(_PERM1), :]
    b2p = b2[jnp.asarray(_PERM2)]

    he1, er1 = _dense1(inputs, W1, al1, ar1, 8, 16)
    er1 = jnp.pad(er1, ((0, NP - N), (0, 0)))
    acc1 = _sc_edge_layer(he1, er1, edges, 8, 128)

    he2, er2 = pl.pallas_call(
        _epi1_body,
        grid=(N // NB,),
        in_specs=[
            pl.BlockSpec((2, NB, 144), lambda i: (0, i, 0)),
            pl.BlockSpec((128,), lambda i: (0,)),
            pl.BlockSpec((128, 32), lambda i: (0, 0)),
            pl.BlockSpec((1, 32), lambda i: (0, 0)),
            pl.BlockSpec((1, 32), lambda i: (0, 0)),
            pl.BlockSpec((16, 128), lambda i: (0, 0)),
        ],
        out_specs=(
            pl.BlockSpec((NB, 64), lambda i: (i, 0)),
            pl.BlockSpec((NB, 32), lambda i: (i, 0)),
        ),
        out_shape=(
            jax.ShapeDtypeStruct((N, 64), jnp.bfloat16),
            jax.ShapeDtypeStruct((N, 32), jnp.bfloat16),
        ),
    )(acc1, b1p, W2p, al2, ar2, jnp.asarray(_MSEL))

    er2 = jnp.pad(er2, ((0, NP - N), (0, 0)))
    acc2 = _sc_edge_layer(he2, er2, edges, 1, 32)

    out = pl.pallas_call(
        _epi2_body,
        grid=(N // NB,),
        in_specs=[
            pl.BlockSpec((2, NB, 48), lambda i: (0, i, 0)),
            pl.BlockSpec((32,), lambda i: (0,)),
            pl.BlockSpec((32, 32), lambda i: (0, 0)),
        ],
        out_specs=pl.BlockSpec((NB, 32), lambda i: (i, 0)),
        out_shape=jax.ShapeDtypeStruct((N, 32), jnp.float32),
    )(acc2, b2p, jnp.asarray(_P32))
    return out
